# baseline (speedup 1.0000x reference)
import jax
import jax.numpy as jnp
from jax import lax
from jax.experimental import pallas as pl
from jax.experimental.pallas import tpu as pltpu
from jax.experimental.pallas import tpu_sc as plsc

N = 10000
E = 320000
D = 128
DE = 16

NC = 2
NS = 16
L = 16
NW = NC * NS
BE = 32
EPW = 5056
EP = EPW * NW
EH = E // 2
NIT = EPW // BE
NPAD = 10240
RPT = NPAD // NS

BN = 1024
BNC = 1000
BEB = 5056


def _proj_body(x_ref, w_ref, b_ref, xk_ref, xq_ref, xv_ref, sk_ref):
    acc = jnp.dot(x_ref[...], w_ref[...], preferred_element_type=jnp.float32)
    xk_ref[...] = acc[:, 0:D]
    xq_ref[...] = acc[:, D:2 * D]
    xv_ref[...] = acc[:, 2 * D:3 * D]
    sk_ref[...] = acc[:, 3 * D:4 * D] + b_ref[...]


def _edge_body(ea_ref, wg_ref, wv_ref, bg_ref, bv_ref, eg_ref, ev_ref):
    ea = ea_ref[...]
    eg_ref[...] = jnp.dot(ea, wg_ref[...], preferred_element_type=jnp.float32) + bg_ref[...]
    ev_ref[...] = jnp.dot(ea, wv_ref[...], preferred_element_type=jnp.float32) + bv_ref[...]


def _comb_body(p_ref, q_ref, s_ref, o_ref):
    o_ref[...] = p_ref[0] + p_ref[1] + q_ref[0] + q_ref[1] + s_ref[...]


def _sc_body(xk_hbm, xq_hbm, xv_hbm, eg_hbm, ev_hbm, src_hbm, dst_hbm, out_hbm,
             di0, si0, kb0, qb0, vb0, gb0, eb0,
             di1, si1, kb1, qb1, vb1, gb1, eb1,
             shared, semi0, semd0, semi1, semd1):
    core = lax.axis_index("c")
    sid = lax.axis_index("s")
    wid = core * NS + sid
    ebase = wid * EPW

    sets = (
        (di0, si0, kb0, qb0, vb0, gb0, eb0, semi0, semd0),
        (di1, si1, kb1, qb1, vb1, gb1, eb1, semi1, semd1),
    )

    def zrow(r, carry):
        for c in range(D // L):
            kb0[r, pl.ds(c * L, L)] = jnp.zeros((L,), jnp.float32)
        return carry

    lax.fori_loop(0, BE, zrow, 0)
    r0 = sid * RPT
    for j in range(RPT // BE):
        pltpu.sync_copy(kb0, shared.at[pl.ds(r0 + j * BE, BE)])
    plsc.subcore_barrier()

    def batch_off(j):
        return ebase + lax.rem(j, NIT) * BE

    def issue_idx(j, s):
        e0 = batch_off(j)
        pltpu.async_copy(dst_hbm.at[pl.ds(e0, BE)], s[0], s[7])
        pltpu.async_copy(src_hbm.at[pl.ds(e0, BE)], s[1], s[7])

    def wait_idx(s):
        pltpu.make_async_copy(dst_hbm.at[pl.ds(0, BE)], s[0], s[7]).wait()
        pltpu.make_async_copy(src_hbm.at[pl.ds(0, BE)], s[1], s[7]).wait()

    def issue_data(j, s):
        e0 = batch_off(j)
        pltpu.async_copy(eg_hbm.at[pl.ds(e0, BE)], s[5], s[8])
        pltpu.async_copy(ev_hbm.at[pl.ds(e0, BE)], s[6], s[8])
        pltpu.async_copy(xk_hbm.at[s[0]], s[2], s[8])
        pltpu.async_copy(xq_hbm.at[s[1]], s[3], s[8])
        pltpu.async_copy(xv_hbm.at[s[1]], s[4], s[8])

    def wait_data(s):
        pltpu.make_async_copy(eg_hbm.at[pl.ds(0, BE)], s[5], s[8]).wait()
        pltpu.make_async_copy(ev_hbm.at[pl.ds(0, BE)], s[6], s[8]).wait()
        pltpu.make_async_copy(xk_hbm.at[s[0]], s[2], s[8]).wait()
        pltpu.make_async_copy(xq_hbm.at[s[1]], s[3], s[8]).wait()
        pltpu.make_async_copy(xv_hbm.at[s[1]], s[4], s[8]).wait()

    def compute_scatter(s):
        didx, kb, qb, vb, gb, eb = s[0], s[2], s[3], s[4], s[5], s[6]

        def edge(e, carry):
            for c in range(D // L):
                sl = pl.ds(L * c, L)
                z = kb[e, sl] + qb[e, sl] + gb[e, sl]
                g = 1.0 / (1.0 + jnp.exp(-z))
                kb[e, sl] = g * (vb[e, sl] + eb[e, sl])
            return carry

        lax.fori_loop(0, BE, edge, 0)
        pltpu.sync_copy(kb, shared.at[didx], add=True)

    def body(j, cur, nxt):
        wait_idx(nxt)
        issue_data(j + 1, nxt)
        wait_data(cur)
        compute_scatter(cur)
        issue_idx(j + 2, cur)

    issue_idx(0, sets[0])
    issue_idx(1, sets[1])
    wait_idx(sets[0])
    issue_data(0, sets[0])

    def loop_body(jj, carry):
        body(2 * jj, sets[0], sets[1])
        body(2 * jj + 1, sets[1], sets[0])
        return carry

    lax.fori_loop(0, NIT // 2, loop_body, 0)

    wait_data(sets[NIT % 2])
    wait_idx(sets[(NIT + 1) % 2])
    plsc.subcore_barrier()

    for j in range(RPT // BE):
        rr = pl.ds(r0 + j * BE, BE)
        pltpu.sync_copy(shared.at[rr], kb0)
        pltpu.sync_copy(kb0, out_hbm.at[core, rr])


_sc_call = pl.kernel(
    _sc_body,
    out_type=jax.ShapeDtypeStruct((NC, NPAD, D), jnp.float32),
    mesh=plsc.VectorSubcoreMesh(core_axis_name="c", subcore_axis_name="s"),
    scratch_types=(
        [
            pltpu.VMEM((BE,), jnp.int32),
            pltpu.VMEM((BE,), jnp.int32),
            pltpu.VMEM((BE, D), jnp.float32),
            pltpu.VMEM((BE, D), jnp.float32),
            pltpu.VMEM((BE, D), jnp.float32),
            pltpu.VMEM((BE, D), jnp.float32),
            pltpu.VMEM((BE, D), jnp.float32),
        ] * 2
        + [
            pltpu.VMEM_SHARED((NPAD, D), jnp.float32),
            pltpu.SemaphoreType.DMA,
            pltpu.SemaphoreType.DMA,
            pltpu.SemaphoreType.DMA,
            pltpu.SemaphoreType.DMA,
        ]
    ),
)


def _edge_proj(ea_half):
    wg_bg = _edge_proj.consts
    wg, wv, bg, bv = wg_bg
    return pl.pallas_call(
        _edge_body,
        grid=(EP // BEB,),
        in_specs=[
            pl.BlockSpec((BEB, DE), lambda i: (i, 0)),
            pl.BlockSpec((DE, D), lambda i: (0, 0)),
            pl.BlockSpec((DE, D), lambda i: (0, 0)),
            pl.BlockSpec((1, D), lambda i: (0, 0)),
            pl.BlockSpec((1, D), lambda i: (0, 0)),
        ],
        out_specs=[pl.BlockSpec((BEB, D), lambda i: (i, 0))] * 2,
        out_shape=[jax.ShapeDtypeStruct((EP, D), jnp.float32)] * 2,
    )(ea_half, wg, wv, bg, bv)


def kernel(x, edge_index, edge_attr, Wk, bk, Wq, bq, Wv, bv, Wskip, bias):
    w_all = jnp.concatenate([Wk[:D], Wq[:D], Wv[:D], Wskip], axis=1)
    xk, xq, xv, skip = pl.pallas_call(
        _proj_body,
        grid=(NPAD // BN,),
        in_specs=[
            pl.BlockSpec((BN, D), lambda i: (i, 0)),
            pl.BlockSpec((D, 4 * D), lambda i: (0, 0)),
            pl.BlockSpec((1, D), lambda i: (0, 0)),
        ],
        out_specs=[pl.BlockSpec((BN, D), lambda i: (i, 0))] * 4,
        out_shape=[jax.ShapeDtypeStruct((NPAD, D), jnp.float32)] * 4,
    )(x, w_all, bias.reshape(1, D))

    wg = Wk[D:] + Wq[D:]
    bg = bk + bq
    _edge_proj.consts = (wg, Wv[D:], bg.reshape(1, D), bv.reshape(1, D))

    halves = []
    for h in range(2):
        sl = slice(h * EH, (h + 1) * EH)
        src_h = jnp.pad(edge_index[0, sl], (0, EP - EH), constant_values=N)
        dst_h = jnp.pad(edge_index[1, sl], (0, EP - EH), constant_values=N)
        eg_h, ev_h = _edge_proj(edge_attr[sl])
        halves.append((eg_h, ev_h, src_h, dst_h))

    p0 = _sc_call(xk, xq, xv, *halves[0])
    p1 = _sc_call(xk, xq, xv, *halves[1])

    out = pl.pallas_call(
        _comb_body,
        grid=(N // BNC,),
        in_specs=[
            pl.BlockSpec((NC, BNC, D), lambda i: (0, i, 0)),
            pl.BlockSpec((NC, BNC, D), lambda i: (0, i, 0)),
            pl.BlockSpec((BNC, D), lambda i: (i, 0)),
        ],
        out_specs=pl.BlockSpec((BNC, D), lambda i: (i, 0)),
        out_shape=jax.ShapeDtypeStruct((N, D), jnp.float32),
    )(p0, p1, skip)
    return out


# speedup vs baseline: 1.0850x; 1.0850x over previous
import jax
import jax.numpy as jnp
from jax import lax
from jax.experimental import pallas as pl
from jax.experimental.pallas import tpu as pltpu
from jax.experimental.pallas import tpu_sc as plsc

N = 10000
E = 320000
D = 128
DE = 16

NC = 2
NS = 16
L = 16
NW = NC * NS
BE = 32
EPW = 10048
EP = EPW * NW
NIT = EPW // BE
NPAD = 10112
RPT = NPAD // NS
RFC = (RPT // BE) * BE

BN = 1264
BNC = 1000
BEB = 5024


def _proj_body(x_ref, w_ref, b_ref, xk_ref, xq_ref, xv_ref, sk_ref):
    acc = jnp.dot(x_ref[...], w_ref[...], preferred_element_type=jnp.float32)
    xk_ref[...] = acc[:, 0:D]
    xq_ref[...] = acc[:, D:2 * D]
    xv_ref[...] = acc[:, 2 * D:3 * D]
    sk_ref[...] = acc[:, 3 * D:4 * D] + b_ref[...]


def _edge_body(ea_ref, wg_ref, wv_ref, bg_ref, bv_ref, eg_ref, ev_ref):
    ea = ea_ref[...]
    eg_ref[...] = jnp.dot(ea, wg_ref[...], preferred_element_type=jnp.float32) + bg_ref[...]
    ev_ref[...] = jnp.dot(ea, wv_ref[...], preferred_element_type=jnp.float32) + bv_ref[...]


def _comb_body(p_ref, s_ref, o_ref):
    o_ref[...] = p_ref[0] + p_ref[1] + s_ref[...]


def _sc_body(xk_hbm, xq_hbm, xv_hbm, eg_hbm, ev_hbm, src_hbm, dst_hbm, out_hbm,
             di0, si0, sc0, kb0, qb0, vb0, gb0, eb0, mb0,
             di1, si1, sc1, kb1, qb1, vb1, gb1, eb1, mb1,
             shared, semi0, semd0, sems0, semi1, semd1, sems1):
    core = lax.axis_index("c")
    sid = lax.axis_index("s")
    wid = core * NS + sid
    ebase = wid * EPW

    sets = (
        (di0, si0, kb0, qb0, vb0, gb0, eb0, semi0, semd0, sc0, mb0, sems0),
        (di1, si1, kb1, qb1, vb1, gb1, eb1, semi1, semd1, sc1, mb1, sems1),
    )

    def zrow(r, carry):
        for c in range(D // L):
            kb0[r, pl.ds(c * L, L)] = jnp.zeros((L,), jnp.float32)
        return carry

    lax.fori_loop(0, BE, zrow, 0)
    r0 = sid * RPT
    for j in range(RPT // BE):
        pltpu.sync_copy(kb0, shared.at[pl.ds(r0 + j * BE, BE)])
    if RPT > RFC:
        pltpu.sync_copy(kb0.at[pl.ds(0, RPT - RFC)],
                        shared.at[pl.ds(r0 + RFC, RPT - RFC)])
    plsc.subcore_barrier()

    def batch_off(j):
        return ebase + lax.rem(j, NIT) * BE

    def issue_idx(j, s):
        e0 = batch_off(j)
        pltpu.async_copy(dst_hbm.at[pl.ds(e0, BE)], s[0], s[7])
        pltpu.async_copy(src_hbm.at[pl.ds(e0, BE)], s[1], s[7])

    def wait_idx(s):
        pltpu.make_async_copy(dst_hbm.at[pl.ds(0, BE)], s[0], s[7]).wait()
        pltpu.make_async_copy(src_hbm.at[pl.ds(0, BE)], s[1], s[7]).wait()

    def issue_data(j, s):
        e0 = batch_off(j)
        pltpu.async_copy(eg_hbm.at[pl.ds(e0, BE)], s[5], s[8])
        pltpu.async_copy(ev_hbm.at[pl.ds(e0, BE)], s[6], s[8])
        pltpu.async_copy(xk_hbm.at[s[0]], s[2], s[8])
        pltpu.async_copy(xq_hbm.at[s[1]], s[3], s[8])
        pltpu.async_copy(xv_hbm.at[s[1]], s[4], s[8])

    def wait_data(s):
        pltpu.make_async_copy(eg_hbm.at[pl.ds(0, BE)], s[5], s[8]).wait()
        pltpu.make_async_copy(ev_hbm.at[pl.ds(0, BE)], s[6], s[8]).wait()
        pltpu.make_async_copy(xk_hbm.at[s[0]], s[2], s[8]).wait()
        pltpu.make_async_copy(xq_hbm.at[s[1]], s[3], s[8]).wait()
        pltpu.make_async_copy(xv_hbm.at[s[1]], s[4], s[8]).wait()

    def wait_scatter(s):
        pltpu.make_async_copy(s[10], shared.at[s[9]], s[11]).wait()

    def compute_scatter(s):
        didx, kb, qb, vb, gb, eb = s[0], s[2], s[3], s[4], s[5], s[6]
        scidx, mb = s[9], s[10]

        def edge(e, carry):
            for c in range(D // L):
                sl = pl.ds(L * c, L)
                z = kb[e, sl] + qb[e, sl] + gb[e, sl]
                g = 1.0 / (1.0 + jnp.exp(-z))
                mb[e, sl] = g * (vb[e, sl] + eb[e, sl])
            return carry

        lax.fori_loop(0, BE, edge, 0)
        for c in range(BE // L):
            scidx[pl.ds(c * L, L)] = didx[pl.ds(c * L, L)]
        pltpu.async_copy(mb, shared.at[scidx], s[11], add=True)

    def body(j, cur, nxt, first=False):
        wait_idx(nxt)
        issue_data(j + 1, nxt)
        wait_data(cur)
        if not first:
            wait_scatter(cur)
        compute_scatter(cur)
        issue_idx(j + 2, cur)

    issue_idx(0, sets[0])
    issue_idx(1, sets[1])
    wait_idx(sets[0])
    issue_data(0, sets[0])
    body(0, sets[0], sets[1], first=True)
    body(1, sets[1], sets[0], first=True)

    def loop_body(jj, carry):
        body(2 * jj + 2, sets[0], sets[1])
        body(2 * jj + 3, sets[1], sets[0])
        return carry

    lax.fori_loop(0, (NIT - 2) // 2, loop_body, 0)

    wait_data(sets[NIT % 2])
    wait_idx(sets[(NIT + 1) % 2])
    wait_scatter(sets[0])
    wait_scatter(sets[1])
    plsc.subcore_barrier()

    for j in range(RPT // BE):
        rr = pl.ds(r0 + j * BE, BE)
        pltpu.sync_copy(shared.at[rr], kb0)
        pltpu.sync_copy(kb0, out_hbm.at[core, rr])
    if RPT > RFC:
        rr = pl.ds(r0 + RFC, RPT - RFC)
        pltpu.sync_copy(shared.at[rr], kb0.at[pl.ds(0, RPT - RFC)])
        pltpu.sync_copy(kb0.at[pl.ds(0, RPT - RFC)], out_hbm.at[core, rr])


_sc_call = pl.kernel(
    _sc_body,
    out_type=jax.ShapeDtypeStruct((NC, NPAD, D), jnp.float32),
    mesh=plsc.VectorSubcoreMesh(core_axis_name="c", subcore_axis_name="s"),
    scratch_types=(
        [
            pltpu.VMEM((BE,), jnp.int32),
            pltpu.VMEM((BE,), jnp.int32),
            pltpu.VMEM((BE,), jnp.int32),
            pltpu.VMEM((BE, D), jnp.float32),
            pltpu.VMEM((BE, D), jnp.float32),
            pltpu.VMEM((BE, D), jnp.float32),
            pltpu.VMEM((BE, D), jnp.float32),
            pltpu.VMEM((BE, D), jnp.float32),
            pltpu.VMEM((BE, D), jnp.float32),
        ] * 2
        + [
            pltpu.VMEM_SHARED((NPAD, D), jnp.float32),
            pltpu.SemaphoreType.DMA,
            pltpu.SemaphoreType.DMA,
            pltpu.SemaphoreType.DMA,
            pltpu.SemaphoreType.DMA,
            pltpu.SemaphoreType.DMA,
            pltpu.SemaphoreType.DMA,
        ]
    ),
)


def kernel(x, edge_index, edge_attr, Wk, bk, Wq, bq, Wv, bv, Wskip, bias):
    w_all = jnp.concatenate([Wk[:D], Wq[:D], Wv[:D], Wskip], axis=1)
    xk, xq, xv, skip = pl.pallas_call(
        _proj_body,
        grid=(NPAD // BN,),
        in_specs=[
            pl.BlockSpec((BN, D), lambda i: (i, 0)),
            pl.BlockSpec((D, 4 * D), lambda i: (0, 0)),
            pl.BlockSpec((1, D), lambda i: (0, 0)),
        ],
        out_specs=[pl.BlockSpec((BN, D), lambda i: (i, 0))] * 4,
        out_shape=[jax.ShapeDtypeStruct((NPAD, D), jnp.float32)] * 4,
    )(x, w_all, bias.reshape(1, D))

    src_pad = jnp.pad(edge_index[0], (0, EP - E), constant_values=N)
    dst_pad = jnp.pad(edge_index[1], (0, EP - E), constant_values=N)
    wg = Wk[D:] + Wq[D:]
    bg = bk + bq
    eg, ev = pl.pallas_call(
        _edge_body,
        grid=(EP // BEB,),
        in_specs=[
            pl.BlockSpec((BEB, DE), lambda i: (i, 0)),
            pl.BlockSpec((DE, D), lambda i: (0, 0)),
            pl.BlockSpec((DE, D), lambda i: (0, 0)),
            pl.BlockSpec((1, D), lambda i: (0, 0)),
            pl.BlockSpec((1, D), lambda i: (0, 0)),
        ],
        out_specs=[pl.BlockSpec((BEB, D), lambda i: (i, 0))] * 2,
        out_shape=[jax.ShapeDtypeStruct((EP, D), jnp.float32)] * 2,
    )(edge_attr, wg, Wv[D:], bg.reshape(1, D), bv.reshape(1, D))

    partial = _sc_call(xk, xq, xv, eg, ev, src_pad, dst_pad)

    out = pl.pallas_call(
        _comb_body,
        grid=(N // BNC,),
        in_specs=[
            pl.BlockSpec((NC, BNC, D), lambda i: (0, i, 0)),
            pl.BlockSpec((BNC, D), lambda i: (i, 0)),
        ],
        out_specs=pl.BlockSpec((BNC, D), lambda i: (i, 0)),
        out_shape=jax.ShapeDtypeStruct((N, D), jnp.float32),
    )(partial, skip)
    return out
